# Initial kernel scaffold; baseline (speedup 1.0000x reference)
import jax
import jax.numpy as jnp
from jax.experimental import pallas as pl


def kernel(x, edge_index, W1, att1, W2, att2):
    raise NotImplementedError("write your pallas kernel here")


# speedup vs baseline: 1.0162x; 1.0162x over previous
import jax
import jax.numpy as jnp
from jax.experimental import pallas as pl

N_NODES = 10000
HEADS = 8
HIDDEN = 8
N_CLASSES = 16


def _mm_body(x_ref, w_ref, o_ref):
    o_ref[...] = jnp.dot(x_ref[...], w_ref[...], preferred_element_type=jnp.float32)


def _matmul(x, w):
    return pl.pallas_call(
        _mm_body,
        out_shape=jax.ShapeDtypeStruct((x.shape[0], w.shape[1]), jnp.float32),
    )(x, w)


def _gat_layer(x, edge_index, W, att, heads, out_channels, concat):
    N = x.shape[0]
    loop = jnp.arange(N, dtype=edge_index.dtype)
    ei = jnp.concatenate([edge_index, jnp.stack([loop, loop], axis=0)], axis=1)
    src, dst = ei[0], ei[1]
    h = _matmul(x, W).reshape(-1, heads, out_channels)
    x_j = h[src]
    x_i = h[dst]
    alpha = (jnp.concatenate([x_i, x_j], axis=-1) * att).sum(axis=-1)
    alpha = jax.nn.leaky_relu(alpha, 0.2)
    amax = jax.ops.segment_max(alpha, dst, num_segments=N)
    amax = jnp.where(jnp.isfinite(amax), amax, 0.0)
    alpha = jnp.exp(alpha - amax[dst])
    denom = jax.ops.segment_sum(alpha, dst, num_segments=N)
    alpha = alpha / (denom[dst] + 1e-16)
    msg = x_j * alpha[:, :, None]
    out = jax.ops.segment_sum(msg, dst, num_segments=N)
    if concat:
        out = out.reshape(N, heads * out_channels)
    else:
        out = out.mean(axis=1)
    return out


def kernel(x, edge_index, W1, att1, W2, att2):
    h = _gat_layer(x, edge_index, W1, att1, HEADS, HIDDEN, True)
    h = jax.nn.elu(h)
    h = _gat_layer(h, edge_index, W2, att2, 1, N_CLASSES, False)
    return jax.nn.log_softmax(h, axis=1)


# speedup vs baseline: 39.9792x; 39.3438x over previous
import functools

import jax
import jax.numpy as jnp
from jax import lax
from jax.experimental import pallas as pl
from jax.experimental.pallas import tpu as pltpu
from jax.experimental.pallas import tpu_sc as plsc

N = 10000
E = 320000
HEADS = 8
HID = 8
NCLS = 16

CORES = 2
SUBCORES = 16
LANES = 16
G = 80
EPT = E // (CORES * SUBCORES)
GROUPS = EPT // G
ZCHUNK = 624
ZTAIL = N - ZCHUNK * SUBCORES

R1 = 80
R2 = 32


def _tc1_body(x_ref, w_ref, ai_m_ref, aj_m_ref, rexp_ref,
              table_ref, aitab_ref, init_ref):
    h = jnp.dot(x_ref[...], w_ref[...], preferred_element_type=jnp.float32)
    ai = jnp.dot(h, ai_m_ref[...], preferred_element_type=jnp.float32)
    aj = jnp.dot(h, aj_m_ref[...], preferred_element_type=jnp.float32)
    t = ai + aj
    s = jnp.exp(jnp.maximum(t, 0.2 * t))
    se = jnp.dot(s, rexp_ref[...], preferred_element_type=jnp.float32)
    z8 = jnp.zeros((h.shape[0], 8), jnp.float32)
    table_ref[...] = jnp.concatenate([h, aj, z8], axis=1)
    aitab_ref[...] = jnp.concatenate([ai, z8], axis=1)
    init_ref[...] = jnp.concatenate([se * h, s, z8], axis=1)


def _tc2_body(p0_ref, p1_ref, init_ref_in, w2_ref, ai2_ref, aj2_ref, rexp_ref,
              table_ref, aitab_ref, init_ref):
    acc = p0_ref[...] + p1_ref[...] + init_ref_in[...]
    num = acc[:, :64]
    den = acc[:, 64:72]
    dene = jnp.dot(den, rexp_ref[...], preferred_element_type=jnp.float32)
    g = num / (dene + 1e-16)
    g = jnp.where(g > 0.0, g, jnp.exp(g) - 1.0)
    h2 = jnp.dot(g, w2_ref[...], preferred_element_type=jnp.float32)
    ai2 = jnp.dot(h2, ai2_ref[...], preferred_element_type=jnp.float32)
    aj2 = jnp.dot(h2, aj2_ref[...], preferred_element_type=jnp.float32)
    t = ai2 + aj2
    s = jnp.exp(jnp.maximum(t, 0.2 * t))
    z15 = jnp.zeros((acc.shape[0], 15), jnp.float32)
    table_ref[...] = jnp.concatenate([h2, aj2, z15], axis=1)
    aitab_ref[...] = jnp.concatenate([ai2, z15], axis=1)
    init_ref[...] = jnp.concatenate([h2 * s, s, z15], axis=1)


def _tc3_body(q0_ref, q1_ref, init_ref_in, out_ref):
    acc = q0_ref[...] + q1_ref[...] + init_ref_in[...]
    num = acc[:, :16]
    den = acc[:, 16:17]
    o = num / (den + 1e-16)
    m = jnp.max(o, axis=1, keepdims=True)
    ex = jnp.exp(o - m)
    lse = jnp.log(jnp.sum(ex, axis=1, keepdims=True))
    out_ref[...] = o - m - lse


def _make_sc_edge(rw, heads, chan):
    mesh = plsc.VectorSubcoreMesh(core_axis_name="c", subcore_axis_name="s")

    @functools.partial(
        pl.kernel,
        mesh=mesh,
        compiler_params=pltpu.CompilerParams(
            needs_layout_passes=False, use_tc_tiling_on_sc=False),
        out_type=jax.ShapeDtypeStruct((CORES, N, rw), jnp.float32),
        scratch_types=[
            pltpu.VMEM((G,), jnp.int32),
            pltpu.VMEM((G,), jnp.int32),
            pltpu.VMEM((G, rw), jnp.float32),
            pltpu.VMEM((G, 16), jnp.float32),
            pltpu.VMEM((G, rw), jnp.float32),
            pltpu.VMEM_SHARED((N, rw), jnp.float32),
            pltpu.SemaphoreType.DMA,
            pltpu.SemaphoreType.DMA,
        ],
    )
    def edge_kernel(table_hbm, aitab_hbm, src_hbm, dst_hbm, zeros_hbm, out_hbm,
                    src_v, dst_v, rows_v, ai_v, msg_v, accum, sem1, sem2):
        c = lax.axis_index("c")
        sid = lax.axis_index("s")
        z0 = sid * ZCHUNK
        pltpu.sync_copy(zeros_hbm.at[pl.ds(z0, ZCHUNK)],
                        accum.at[pl.ds(z0, ZCHUNK)])
        @pl.when(sid == 0)
        def _():
            pltpu.sync_copy(zeros_hbm.at[pl.ds(ZCHUNK * SUBCORES, ZTAIL)],
                            accum.at[pl.ds(ZCHUNK * SUBCORES, ZTAIL)])
        plsc.subcore_barrier()

        base = (c * SUBCORES + sid) * EPT

        def group(gi, carry):
            e0 = pl.multiple_of(base + gi * G, G)
            pltpu.sync_copy(src_hbm.at[pl.ds(e0, G)], src_v)
            pltpu.sync_copy(dst_hbm.at[pl.ds(e0, G)], dst_v)
            cp1 = pltpu.async_copy(table_hbm.at[src_v], rows_v, sem1)
            cp2 = pltpu.async_copy(aitab_hbm.at[dst_v], ai_v, sem2)
            cp1.wait()
            cp2.wait()
            for sub in range(G // LANES):
                rid = lax.iota(jnp.int32, 16) + (sub * LANES)
                for h in range(heads):
                    acol = jnp.full((16,), heads * chan + h, jnp.int32)
                    ajv = plsc.load_gather(rows_v, [rid, acol])
                    aiv = plsc.load_gather(ai_v, [rid, jnp.full((16,), h, jnp.int32)])
                    t = aiv + ajv
                    sv = jnp.exp(jnp.maximum(t, 0.2 * t))
                    plsc.store_scatter(msg_v, [rid, acol], sv)
                    for ch in range(chan):
                        fcol = jnp.full((16,), h * chan + ch, jnp.int32)
                        fv = plsc.load_gather(rows_v, [rid, fcol])
                        plsc.store_scatter(msg_v, [rid, fcol], sv * fv)
            pltpu.sync_copy(msg_v, accum.at[dst_v], add=True)
            return carry

        lax.fori_loop(0, GROUPS, group, 0)
        plsc.subcore_barrier()
        pltpu.sync_copy(accum.at[pl.ds(z0, ZCHUNK)],
                        out_hbm.at[c, pl.ds(z0, ZCHUNK)])
        @pl.when(sid == 0)
        def _():
            pltpu.sync_copy(accum.at[pl.ds(ZCHUNK * SUBCORES, ZTAIL)],
                            out_hbm.at[c, pl.ds(ZCHUNK * SUBCORES, ZTAIL)])

    return edge_kernel


_EDGE1 = _make_sc_edge(R1, HEADS, HID)
_EDGE2 = _make_sc_edge(R2, 1, NCLS)


def kernel(x, edge_index, W1, att1, W2, att2):
    src = edge_index[0]
    dst = edge_index[1]
    eye = jnp.eye(HEADS, dtype=jnp.float32)
    ai_m1 = (att1[0, :, :HID][:, :, None] * eye[:, None, :]).reshape(HEADS * HID, HEADS)
    aj_m1 = (att1[0, :, HID:][:, :, None] * eye[:, None, :]).reshape(HEADS * HID, HEADS)
    rexp = jnp.repeat(eye, HID, axis=1)
    ai_m2 = att2[0, 0, :NCLS].reshape(NCLS, 1)
    aj_m2 = att2[0, 0, NCLS:].reshape(NCLS, 1)

    table1, aitab1, init1 = pl.pallas_call(
        _tc1_body,
        out_shape=(
            jax.ShapeDtypeStruct((N, R1), jnp.float32),
            jax.ShapeDtypeStruct((N, 16), jnp.float32),
            jax.ShapeDtypeStruct((N, R1), jnp.float32),
        ),
    )(x, W1, ai_m1, aj_m1, rexp)

    z1 = jnp.zeros((N, R1), jnp.float32)
    partial1 = _EDGE1(table1, aitab1, src, dst, z1)

    table2, aitab2, init2 = pl.pallas_call(
        _tc2_body,
        out_shape=(
            jax.ShapeDtypeStruct((N, R2), jnp.float32),
            jax.ShapeDtypeStruct((N, 16), jnp.float32),
            jax.ShapeDtypeStruct((N, R2), jnp.float32),
        ),
    )(partial1[0], partial1[1], init1, W2, ai_m2, aj_m2, rexp)

    z2 = jnp.zeros((N, R2), jnp.float32)
    partial2 = _EDGE2(table2, aitab2, src, dst, z2)

    return pl.pallas_call(
        _tc3_body,
        out_shape=jax.ShapeDtypeStruct((N, NCLS), jnp.float32),
    )(partial2[0], partial2[1], init2)


# speedup vs baseline: 55.2497x; 1.3820x over previous
import functools

import jax
import jax.numpy as jnp
from jax import lax
from jax.experimental import pallas as pl
from jax.experimental.pallas import tpu as pltpu
from jax.experimental.pallas import tpu_sc as plsc

N = 10000
E = 320000
HEADS = 8
HID = 8
NCLS = 16

CORES = 2
SUBCORES = 16
LANES = 16
G = 80
EPT = E // (CORES * SUBCORES)
GROUPS = EPT // G
ZCHUNK = 624
ZTAIL = N - ZCHUNK * SUBCORES

R1 = 80
R2 = 32


def _tc1_body(x_ref, w_ref, ai_m_ref, aj_m_ref, rexp_ref,
              table_ref, aitab_ref, init_ref):
    h = jnp.dot(x_ref[...], w_ref[...], preferred_element_type=jnp.float32)
    ai = jnp.dot(h, ai_m_ref[...], preferred_element_type=jnp.float32)
    aj = jnp.dot(h, aj_m_ref[...], preferred_element_type=jnp.float32)
    t = ai + aj
    s = jnp.exp(jnp.maximum(t, 0.2 * t))
    se = jnp.dot(s, rexp_ref[...], preferred_element_type=jnp.float32)
    z8 = jnp.zeros((h.shape[0], 8), jnp.float32)
    table_ref[...] = jnp.concatenate([h, aj, z8], axis=1)
    aitab_ref[...] = jnp.concatenate([ai, z8], axis=1)
    init_ref[...] = jnp.concatenate([se * h, s, z8], axis=1)


def _tc2_body(p0_ref, p1_ref, init_ref_in, w2_ref, ai2_ref, aj2_ref, rexp_ref,
              table_ref, aitab_ref, init_ref):
    acc = p0_ref[...] + p1_ref[...] + init_ref_in[...]
    num = acc[:, :64]
    den = acc[:, 64:72]
    dene = jnp.dot(den, rexp_ref[...], preferred_element_type=jnp.float32)
    g = num / (dene + 1e-16)
    g = jnp.where(g > 0.0, g, jnp.exp(g) - 1.0)
    h2 = jnp.dot(g, w2_ref[...], preferred_element_type=jnp.float32)
    ai2 = jnp.dot(h2, ai2_ref[...], preferred_element_type=jnp.float32)
    aj2 = jnp.dot(h2, aj2_ref[...], preferred_element_type=jnp.float32)
    t = ai2 + aj2
    s = jnp.exp(jnp.maximum(t, 0.2 * t))
    z15 = jnp.zeros((acc.shape[0], 15), jnp.float32)
    table_ref[...] = jnp.concatenate([h2, aj2, z15], axis=1)
    aitab_ref[...] = jnp.concatenate([ai2, z15], axis=1)
    init_ref[...] = jnp.concatenate([h2 * s, s, z15], axis=1)


def _tc3_body(q0_ref, q1_ref, init_ref_in, out_ref):
    acc = q0_ref[...] + q1_ref[...] + init_ref_in[...]
    num = acc[:, :16]
    den = acc[:, 16:17]
    o = num / (den + 1e-16)
    m = jnp.max(o, axis=1, keepdims=True)
    ex = jnp.exp(o - m)
    lse = jnp.log(jnp.sum(ex, axis=1, keepdims=True))
    out_ref[...] = o - m - lse


def _make_sc_edge(rw, heads, chan):
    mesh = plsc.VectorSubcoreMesh(core_axis_name="c", subcore_axis_name="s")

    @functools.partial(
        pl.kernel,
        mesh=mesh,
        compiler_params=pltpu.CompilerParams(
            needs_layout_passes=False, use_tc_tiling_on_sc=False),
        out_type=jax.ShapeDtypeStruct((CORES, N, rw), jnp.float32),
        scratch_types=[
            pltpu.VMEM((GROUPS, G), jnp.int32),
            pltpu.VMEM((GROUPS, G), jnp.int32),
            pltpu.VMEM((G, rw), jnp.float32),
            pltpu.VMEM((G, rw), jnp.float32),
            pltpu.VMEM((G, 16), jnp.float32),
            pltpu.VMEM((G, 16), jnp.float32),
            pltpu.VMEM((G, rw), jnp.float32),
            pltpu.VMEM((G, rw), jnp.float32),
            pltpu.VMEM_SHARED((N, rw), jnp.float32),
            pltpu.SemaphoreType.DMA,
            pltpu.SemaphoreType.DMA,
            pltpu.SemaphoreType.DMA,
            pltpu.SemaphoreType.DMA,
            pltpu.SemaphoreType.DMA,
            pltpu.SemaphoreType.DMA,
        ],
    )
    def edge_kernel(table_hbm, aitab_hbm, srcg_hbm, dstg_hbm, zeros_hbm,
                    out_hbm, src_all, dst_all, rows_a, rows_b, ai_a, ai_b,
                    msg_a, msg_b, accum, sem_ta, sem_tb, sem_aa, sem_ab,
                    sem_sa, sem_sb):
        c = lax.axis_index("c")
        sid = lax.axis_index("s")
        z0 = sid * ZCHUNK
        pltpu.sync_copy(zeros_hbm.at[pl.ds(z0, ZCHUNK)],
                        accum.at[pl.ds(z0, ZCHUNK)])

        @pl.when(sid == 0)
        def _():
            pltpu.sync_copy(zeros_hbm.at[pl.ds(ZCHUNK * SUBCORES, ZTAIL)],
                            accum.at[pl.ds(ZCHUNK * SUBCORES, ZTAIL)])

        gbase = (c * SUBCORES + sid) * GROUPS
        pltpu.sync_copy(srcg_hbm.at[pl.ds(gbase, GROUPS)], src_all)
        pltpu.sync_copy(dstg_hbm.at[pl.ds(gbase, GROUPS)], dst_all)
        pltpu.sync_copy(zeros_hbm.at[pl.ds(0, G)], msg_a)
        pltpu.sync_copy(zeros_hbm.at[pl.ds(0, G)], msg_b)
        plsc.subcore_barrier()

        def issue(gi, rows_v, ai_v, sem_t, sem_ai):
            pltpu.async_copy(table_hbm.at[src_all.at[gi]], rows_v, sem_t)
            pltpu.async_copy(aitab_hbm.at[dst_all.at[gi]], ai_v, sem_ai)

        def wait_gathers(gi, rows_v, ai_v, sem_t, sem_ai):
            pltpu.make_async_copy(table_hbm.at[src_all.at[gi]], rows_v, sem_t).wait()
            pltpu.make_async_copy(aitab_hbm.at[dst_all.at[gi]], ai_v, sem_ai).wait()

        def start_scatter(gi, msg_v, sem_s):
            pltpu.async_copy(msg_v, accum.at[dst_all.at[gi]], sem_s, add=True)

        def wait_scatter(gi, msg_v, sem_s):
            pltpu.make_async_copy(msg_v, accum.at[dst_all.at[gi]], sem_s).wait()

        def compute(rows_v, ai_v, msg_v):
            for sub in range(G // LANES):
                rid = lax.iota(jnp.int32, 16) + (sub * LANES)
                for h in range(heads):
                    acol = jnp.full((16,), heads * chan + h, jnp.int32)
                    ajv = plsc.load_gather(rows_v, [rid, acol])
                    aiv = plsc.load_gather(ai_v, [rid, jnp.full((16,), h, jnp.int32)])
                    t = aiv + ajv
                    sv = jnp.exp(jnp.maximum(t, 0.2 * t))
                    plsc.store_scatter(msg_v, [rid, acol], sv)
                    for ch in range(chan):
                        fcol = jnp.full((16,), h * chan + ch, jnp.int32)
                        fv = plsc.load_gather(rows_v, [rid, fcol])
                        plsc.store_scatter(msg_v, [rid, fcol], sv * fv)

        issue(0, rows_a, ai_a, sem_ta, sem_aa)
        issue(1, rows_b, ai_b, sem_tb, sem_ab)
        start_scatter(0, msg_a, sem_sa)
        start_scatter(0, msg_b, sem_sb)

        def pair(p, carry):
            ga = 2 * p
            gb = 2 * p + 1
            wait_gathers(ga, rows_a, ai_a, sem_ta, sem_aa)
            wait_scatter(ga, msg_a, sem_sa)
            compute(rows_a, ai_a, msg_a)
            start_scatter(ga, msg_a, sem_sa)
            issue(ga + 2, rows_a, ai_a, sem_ta, sem_aa)
            wait_gathers(gb, rows_b, ai_b, sem_tb, sem_ab)
            wait_scatter(gb, msg_b, sem_sb)
            compute(rows_b, ai_b, msg_b)
            start_scatter(gb, msg_b, sem_sb)

            @pl.when(gb + 2 < GROUPS)
            def _():
                issue(gb + 2, rows_b, ai_b, sem_tb, sem_ab)

            return carry

        lax.fori_loop(0, (GROUPS - 1) // 2, pair, 0)
        glast = GROUPS - 1
        wait_gathers(glast, rows_a, ai_a, sem_ta, sem_aa)
        wait_scatter(glast - 2, msg_a, sem_sa)
        compute(rows_a, ai_a, msg_a)
        start_scatter(glast, msg_a, sem_sa)
        wait_scatter(glast, msg_a, sem_sa)
        wait_scatter(glast - 1, msg_b, sem_sb)
        plsc.subcore_barrier()
        pltpu.sync_copy(accum.at[pl.ds(z0, ZCHUNK)],
                        out_hbm.at[c, pl.ds(z0, ZCHUNK)])

        @pl.when(sid == 0)
        def _():
            pltpu.sync_copy(accum.at[pl.ds(ZCHUNK * SUBCORES, ZTAIL)],
                            out_hbm.at[c, pl.ds(ZCHUNK * SUBCORES, ZTAIL)])

    return edge_kernel


_EDGE1 = _make_sc_edge(R1, HEADS, HID)
_EDGE2 = _make_sc_edge(R2, 1, NCLS)


def kernel(x, edge_index, W1, att1, W2, att2):
    src = edge_index[0].reshape(CORES * SUBCORES * GROUPS, G)
    dst = edge_index[1].reshape(CORES * SUBCORES * GROUPS, G)
    eye = jnp.eye(HEADS, dtype=jnp.float32)
    ai_m1 = (att1[0, :, :HID][:, :, None] * eye[:, None, :]).reshape(HEADS * HID, HEADS)
    aj_m1 = (att1[0, :, HID:][:, :, None] * eye[:, None, :]).reshape(HEADS * HID, HEADS)
    rexp = jnp.repeat(eye, HID, axis=1)
    ai_m2 = att2[0, 0, :NCLS].reshape(NCLS, 1)
    aj_m2 = att2[0, 0, NCLS:].reshape(NCLS, 1)

    table1, aitab1, init1 = pl.pallas_call(
        _tc1_body,
        out_shape=(
            jax.ShapeDtypeStruct((N, R1), jnp.float32),
            jax.ShapeDtypeStruct((N, 16), jnp.float32),
            jax.ShapeDtypeStruct((N, R1), jnp.float32),
        ),
    )(x, W1, ai_m1, aj_m1, rexp)

    z1 = jnp.zeros((N, R1), jnp.float32)
    partial1 = _EDGE1(table1, aitab1, src, dst, z1)

    table2, aitab2, init2 = pl.pallas_call(
        _tc2_body,
        out_shape=(
            jax.ShapeDtypeStruct((N, R2), jnp.float32),
            jax.ShapeDtypeStruct((N, 16), jnp.float32),
            jax.ShapeDtypeStruct((N, R2), jnp.float32),
        ),
    )(partial1[0], partial1[1], init1, W2, ai_m2, aj_m2, rexp)

    z2 = jnp.zeros((N, R2), jnp.float32)
    partial2 = _EDGE2(table2, aitab2, src, dst, z2)

    return pl.pallas_call(
        _tc3_body,
        out_shape=jax.ShapeDtypeStruct((N, NCLS), jnp.float32),
    )(partial2[0], partial2[1], init2)


# speedup vs baseline: 68.0059x; 1.2309x over previous
import functools

import jax
import jax.numpy as jnp
from jax import lax
from jax.experimental import pallas as pl
from jax.experimental.pallas import tpu as pltpu
from jax.experimental.pallas import tpu_sc as plsc

N = 10000
E = 320000
HEADS = 8
HID = 8
NCLS = 16

CORES = 2
SUBCORES = 16
LANES = 16
G = 80
EPT = E // (CORES * SUBCORES)
GROUPS = EPT // G
ZCHUNK = 624
ZTAIL = N - ZCHUNK * SUBCORES

R1 = 81
R2 = 33
AIW = 17


def _tc1_body(x_ref, w_ref, ai_m_ref, aj_m_ref, rexp_ref,
              table_ref, aitab_ref, init_ref):
    h = jnp.dot(x_ref[...], w_ref[...], preferred_element_type=jnp.float32)
    ai = jnp.dot(h, ai_m_ref[...], preferred_element_type=jnp.float32)
    aj = jnp.dot(h, aj_m_ref[...], preferred_element_type=jnp.float32)
    t = ai + aj
    s = jnp.exp(jnp.maximum(t, 0.2 * t))
    se = jnp.dot(s, rexp_ref[...], preferred_element_type=jnp.float32)
    z9 = jnp.zeros((h.shape[0], 9), jnp.float32)
    table_ref[...] = jnp.concatenate([h, aj, z9], axis=1)
    aitab_ref[...] = jnp.concatenate([ai, z9], axis=1)
    init_ref[...] = jnp.concatenate([se * h, s, z9], axis=1)


def _tc2_body(p0_ref, p1_ref, init_ref_in, w2_ref, ai2_ref, aj2_ref, rexp_ref,
              table_ref, aitab_ref, init_ref):
    acc = p0_ref[...] + p1_ref[...] + init_ref_in[...]
    num = acc[:, :64]
    den = acc[:, 64:72]
    dene = jnp.dot(den, rexp_ref[...], preferred_element_type=jnp.float32)
    g = num / (dene + 1e-16)
    g = jnp.where(g > 0.0, g, jnp.exp(g) - 1.0)
    h2 = jnp.dot(g, w2_ref[...], preferred_element_type=jnp.float32)
    ai2 = jnp.dot(h2, ai2_ref[...], preferred_element_type=jnp.float32)
    aj2 = jnp.dot(h2, aj2_ref[...], preferred_element_type=jnp.float32)
    t = ai2 + aj2
    s = jnp.exp(jnp.maximum(t, 0.2 * t))
    z16 = jnp.zeros((acc.shape[0], 16), jnp.float32)
    z15 = z16[:, :15]
    table_ref[...] = jnp.concatenate([h2, aj2, z16], axis=1)
    aitab_ref[...] = jnp.concatenate([ai2, z16], axis=1)
    init_ref[...] = jnp.concatenate([h2 * s, s, z16], axis=1)


def _tc3_body(q0_ref, q1_ref, init_ref_in, out_ref):
    acc = q0_ref[...] + q1_ref[...] + init_ref_in[...]
    num = acc[:, :16]
    den = acc[:, 16:17]
    o = num / (den + 1e-16)
    m = jnp.max(o, axis=1, keepdims=True)
    ex = jnp.exp(o - m)
    lse = jnp.log(jnp.sum(ex, axis=1, keepdims=True))
    out_ref[...] = o - m - lse


def _make_sc_edge(rw, heads, chan):
    mesh = plsc.VectorSubcoreMesh(core_axis_name="c", subcore_axis_name="s")

    @functools.partial(
        pl.kernel,
        mesh=mesh,
        compiler_params=pltpu.CompilerParams(
            needs_layout_passes=False, use_tc_tiling_on_sc=False),
        out_type=jax.ShapeDtypeStruct((CORES, N, rw), jnp.float32),
        scratch_types=[
            pltpu.VMEM((GROUPS, G), jnp.int32),
            pltpu.VMEM((GROUPS, G), jnp.int32),
            pltpu.VMEM((G, rw), jnp.float32),
            pltpu.VMEM((G, rw), jnp.float32),
            pltpu.VMEM((G, AIW), jnp.float32),
            pltpu.VMEM((G, AIW), jnp.float32),
            pltpu.VMEM((G, rw), jnp.float32),
            pltpu.VMEM((G, rw), jnp.float32),
            pltpu.VMEM_SHARED((N, rw), jnp.float32),
            pltpu.SemaphoreType.DMA,
            pltpu.SemaphoreType.DMA,
            pltpu.SemaphoreType.DMA,
            pltpu.SemaphoreType.DMA,
            pltpu.SemaphoreType.DMA,
            pltpu.SemaphoreType.DMA,
        ],
    )
    def edge_kernel(table_hbm, aitab_hbm, srcg_hbm, dstg_hbm, zeros_hbm,
                    out_hbm, src_all, dst_all, rows_a, rows_b, ai_a, ai_b,
                    msg_a, msg_b, accum, sem_ta, sem_tb, sem_aa, sem_ab,
                    sem_sa, sem_sb):
        c = lax.axis_index("c")
        sid = lax.axis_index("s")
        z0 = sid * ZCHUNK
        pltpu.sync_copy(zeros_hbm.at[pl.ds(z0, ZCHUNK)],
                        accum.at[pl.ds(z0, ZCHUNK)])

        @pl.when(sid == 0)
        def _():
            pltpu.sync_copy(zeros_hbm.at[pl.ds(ZCHUNK * SUBCORES, ZTAIL)],
                            accum.at[pl.ds(ZCHUNK * SUBCORES, ZTAIL)])

        gbase = (c * SUBCORES + sid) * GROUPS
        pltpu.sync_copy(srcg_hbm.at[pl.ds(gbase, GROUPS)], src_all)
        pltpu.sync_copy(dstg_hbm.at[pl.ds(gbase, GROUPS)], dst_all)
        pltpu.sync_copy(zeros_hbm.at[pl.ds(0, G)], msg_a)
        pltpu.sync_copy(zeros_hbm.at[pl.ds(0, G)], msg_b)
        plsc.subcore_barrier()

        def issue(gi, rows_v, ai_v, sem_t, sem_ai):
            pltpu.async_copy(table_hbm.at[src_all.at[gi]], rows_v, sem_t)
            pltpu.async_copy(aitab_hbm.at[dst_all.at[gi]], ai_v, sem_ai)

        def wait_gathers(gi, rows_v, ai_v, sem_t, sem_ai):
            pltpu.make_async_copy(table_hbm.at[src_all.at[gi]], rows_v, sem_t).wait()
            pltpu.make_async_copy(aitab_hbm.at[dst_all.at[gi]], ai_v, sem_ai).wait()

        def start_scatter(gi, msg_v, sem_s):
            pltpu.async_copy(msg_v, accum.at[dst_all.at[gi]], sem_s, add=True)

        def wait_scatter(gi, msg_v, sem_s):
            pltpu.make_async_copy(msg_v, accum.at[dst_all.at[gi]], sem_s).wait()

        def compute(rows_v, ai_v, msg_v):
            for sub in range(G // LANES):
                rid = lax.iota(jnp.int32, 16) + (sub * LANES)
                for h in range(heads):
                    acol = jnp.full((16,), heads * chan + h, jnp.int32)
                    ajv = plsc.load_gather(rows_v, [rid, acol])
                    aiv = plsc.load_gather(ai_v, [rid, jnp.full((16,), h, jnp.int32)])
                    t = aiv + ajv
                    sv = jnp.exp(jnp.maximum(t, 0.2 * t))
                    plsc.store_scatter(msg_v, [rid, acol], sv)
                    for ch in range(chan):
                        fcol = jnp.full((16,), h * chan + ch, jnp.int32)
                        fv = plsc.load_gather(rows_v, [rid, fcol])
                        plsc.store_scatter(msg_v, [rid, fcol], sv * fv)

        issue(0, rows_a, ai_a, sem_ta, sem_aa)
        issue(1, rows_b, ai_b, sem_tb, sem_ab)
        start_scatter(0, msg_a, sem_sa)
        start_scatter(0, msg_b, sem_sb)

        def pair(p, carry):
            ga = 2 * p
            gb = 2 * p + 1
            wait_gathers(ga, rows_a, ai_a, sem_ta, sem_aa)
            wait_scatter(ga, msg_a, sem_sa)
            compute(rows_a, ai_a, msg_a)
            start_scatter(ga, msg_a, sem_sa)
            issue(ga + 2, rows_a, ai_a, sem_ta, sem_aa)
            wait_gathers(gb, rows_b, ai_b, sem_tb, sem_ab)
            wait_scatter(gb, msg_b, sem_sb)
            compute(rows_b, ai_b, msg_b)
            start_scatter(gb, msg_b, sem_sb)

            @pl.when(gb + 2 < GROUPS)
            def _():
                issue(gb + 2, rows_b, ai_b, sem_tb, sem_ab)

            return carry

        lax.fori_loop(0, (GROUPS - 1) // 2, pair, 0)
        glast = GROUPS - 1
        wait_gathers(glast, rows_a, ai_a, sem_ta, sem_aa)
        wait_scatter(glast - 2, msg_a, sem_sa)
        compute(rows_a, ai_a, msg_a)
        start_scatter(glast, msg_a, sem_sa)
        wait_scatter(glast, msg_a, sem_sa)
        wait_scatter(glast - 1, msg_b, sem_sb)
        plsc.subcore_barrier()
        pltpu.sync_copy(accum.at[pl.ds(z0, ZCHUNK)],
                        out_hbm.at[c, pl.ds(z0, ZCHUNK)])

        @pl.when(sid == 0)
        def _():
            pltpu.sync_copy(accum.at[pl.ds(ZCHUNK * SUBCORES, ZTAIL)],
                            out_hbm.at[c, pl.ds(ZCHUNK * SUBCORES, ZTAIL)])

    return edge_kernel


_EDGE1 = _make_sc_edge(R1, HEADS, HID)
_EDGE2 = _make_sc_edge(R2, 1, NCLS)


def kernel(x, edge_index, W1, att1, W2, att2):
    src = edge_index[0].reshape(CORES * SUBCORES * GROUPS, G)
    dst = edge_index[1].reshape(CORES * SUBCORES * GROUPS, G)
    eye = jnp.eye(HEADS, dtype=jnp.float32)
    ai_m1 = (att1[0, :, :HID][:, :, None] * eye[:, None, :]).reshape(HEADS * HID, HEADS)
    aj_m1 = (att1[0, :, HID:][:, :, None] * eye[:, None, :]).reshape(HEADS * HID, HEADS)
    rexp = jnp.repeat(eye, HID, axis=1)
    ai_m2 = att2[0, 0, :NCLS].reshape(NCLS, 1)
    aj_m2 = att2[0, 0, NCLS:].reshape(NCLS, 1)

    table1, aitab1, init1 = pl.pallas_call(
        _tc1_body,
        out_shape=(
            jax.ShapeDtypeStruct((N, R1), jnp.float32),
            jax.ShapeDtypeStruct((N, AIW), jnp.float32),
            jax.ShapeDtypeStruct((N, R1), jnp.float32),
        ),
    )(x, W1, ai_m1, aj_m1, rexp)

    z1 = jnp.zeros((N, R1), jnp.float32)
    partial1 = _EDGE1(table1, aitab1, src, dst, z1)

    table2, aitab2, init2 = pl.pallas_call(
        _tc2_body,
        out_shape=(
            jax.ShapeDtypeStruct((N, R2), jnp.float32),
            jax.ShapeDtypeStruct((N, AIW), jnp.float32),
            jax.ShapeDtypeStruct((N, R2), jnp.float32),
        ),
    )(partial1[0], partial1[1], init1, W2, ai_m2, aj_m2, rexp)

    z2 = jnp.zeros((N, R2), jnp.float32)
    partial2 = _EDGE2(table2, aitab2, src, dst, z2)

    return pl.pallas_call(
        _tc3_body,
        out_shape=jax.ShapeDtypeStruct((N, NCLS), jnp.float32),
    )(partial2[0], partial2[1], init2)


# speedup vs baseline: 173.2738x; 2.5479x over previous
import functools

import jax
import jax.numpy as jnp
from jax import lax
from jax.experimental import pallas as pl
from jax.experimental.pallas import tpu as pltpu
from jax.experimental.pallas import tpu_sc as plsc

N = 10000
E = 320000
HEADS = 8
HID = 8
NCLS = 16

CORES = 2
SUBCORES = 16
LANES = 16
G = 80
EPT = E // (CORES * SUBCORES)
GROUPS = EPT // G
ZCHUNK = 624
ZTAIL = N - ZCHUNK * SUBCORES

R1 = 80
R2 = 32
AIW = 16


def _tc1_body(x_ref, w_ref, ai_m_ref, aj_m_ref, rexp_ref,
              table_ref, aitab_ref, init_ref):
    h = jnp.dot(x_ref[...], w_ref[...], preferred_element_type=jnp.float32)
    ai = jnp.dot(h, ai_m_ref[...], preferred_element_type=jnp.float32)
    aj = jnp.dot(h, aj_m_ref[...], preferred_element_type=jnp.float32)
    t = ai + aj
    s = jnp.exp(jnp.maximum(t, 0.2 * t))
    se = jnp.dot(s, rexp_ref[...], preferred_element_type=jnp.float32)
    z8 = jnp.zeros((h.shape[0], 8), jnp.float32)
    table_ref[...] = jnp.concatenate([h, aj, z8], axis=1)
    aitab_ref[...] = jnp.concatenate([ai, z8], axis=1)
    init_ref[...] = jnp.concatenate([se * h, s, z8], axis=1)


def _tc2_body(p0_ref, p1_ref, init_ref_in, w2_ref, ai2_ref, aj2_ref, rexp_ref,
              table_ref, aitab_ref, init_ref):
    acc = p0_ref[...] + p1_ref[...] + init_ref_in[...]
    num = acc[:, :64]
    den = acc[:, 64:72]
    dene = jnp.dot(den, rexp_ref[...], preferred_element_type=jnp.float32)
    g = num / (dene + 1e-16)
    g = jnp.where(g > 0.0, g, jnp.exp(g) - 1.0)
    h2 = jnp.dot(g, w2_ref[...], preferred_element_type=jnp.float32)
    ai2 = jnp.dot(h2, ai2_ref[...], preferred_element_type=jnp.float32)
    aj2 = jnp.dot(h2, aj2_ref[...], preferred_element_type=jnp.float32)
    t = ai2 + aj2
    s = jnp.exp(jnp.maximum(t, 0.2 * t))
    z15 = jnp.zeros((acc.shape[0], 15), jnp.float32)
    table_ref[...] = jnp.concatenate([h2, aj2, z15], axis=1)
    aitab_ref[...] = jnp.concatenate([ai2, z15], axis=1)
    init_ref[...] = jnp.concatenate([h2 * s, s, z15], axis=1)


def _tc3_body(q0_ref, q1_ref, init_ref_in, out_ref):
    acc = q0_ref[...] + q1_ref[...] + init_ref_in[...]
    num = acc[:, :16]
    den = acc[:, 16:17]
    o = num / (den + 1e-16)
    m = jnp.max(o, axis=1, keepdims=True)
    ex = jnp.exp(o - m)
    lse = jnp.log(jnp.sum(ex, axis=1, keepdims=True))
    out_ref[...] = o - m - lse


def _make_sc_edge(rw, heads, chan):
    mesh = plsc.VectorSubcoreMesh(core_axis_name="c", subcore_axis_name="s")

    @functools.partial(
        pl.kernel,
        mesh=mesh,
        compiler_params=pltpu.CompilerParams(
            needs_layout_passes=False, use_tc_tiling_on_sc=False),
        out_type=jax.ShapeDtypeStruct((CORES, N, rw), jnp.float32),
        scratch_types=[
            pltpu.VMEM((GROUPS, G), jnp.int32),
            pltpu.VMEM((GROUPS, G), jnp.int32),
            pltpu.VMEM((G, rw), jnp.float32),
            pltpu.VMEM((G, rw), jnp.float32),
            pltpu.VMEM((G, AIW), jnp.float32),
            pltpu.VMEM((G, AIW), jnp.float32),
            pltpu.VMEM((G, rw), jnp.float32),
            pltpu.VMEM((G, rw), jnp.float32),
            pltpu.VMEM_SHARED((N, rw), jnp.float32),
            pltpu.SemaphoreType.DMA,
            pltpu.SemaphoreType.DMA,
            pltpu.SemaphoreType.DMA,
            pltpu.SemaphoreType.DMA,
            pltpu.SemaphoreType.DMA,
            pltpu.SemaphoreType.DMA,
        ],
    )
    def edge_kernel(table_hbm, aitab_hbm, srcg_hbm, dstg_hbm, zeros_hbm,
                    out_hbm, src_all, dst_all, rows_a, rows_b, ai_a, ai_b,
                    msg_a, msg_b, accum, sem_ta, sem_tb, sem_aa, sem_ab,
                    sem_sa, sem_sb):
        c = lax.axis_index("c")
        sid = lax.axis_index("s")
        z0 = sid * ZCHUNK
        pltpu.sync_copy(zeros_hbm.at[pl.ds(z0, ZCHUNK)],
                        accum.at[pl.ds(z0, ZCHUNK)])

        @pl.when(sid == 0)
        def _():
            pltpu.sync_copy(zeros_hbm.at[pl.ds(ZCHUNK * SUBCORES, ZTAIL)],
                            accum.at[pl.ds(ZCHUNK * SUBCORES, ZTAIL)])

        gbase = (c * SUBCORES + sid) * GROUPS
        pltpu.sync_copy(srcg_hbm.at[pl.ds(gbase, GROUPS)], src_all)
        pltpu.sync_copy(dstg_hbm.at[pl.ds(gbase, GROUPS)], dst_all)
        pltpu.sync_copy(zeros_hbm.at[pl.ds(0, G)], msg_a)
        pltpu.sync_copy(zeros_hbm.at[pl.ds(0, G)], msg_b)
        plsc.subcore_barrier()

        def issue(gi, rows_v, ai_v, sem_t, sem_ai):
            pltpu.async_copy(table_hbm.at[src_all.at[gi]], rows_v, sem_t)
            pltpu.async_copy(aitab_hbm.at[dst_all.at[gi]], ai_v, sem_ai)

        def wait_gathers(gi, rows_v, ai_v, sem_t, sem_ai):
            pltpu.make_async_copy(table_hbm.at[src_all.at[gi]], rows_v, sem_t).wait()
            pltpu.make_async_copy(aitab_hbm.at[dst_all.at[gi]], ai_v, sem_ai).wait()

        def start_scatter(gi, msg_v, sem_s):
            pltpu.async_copy(msg_v, accum.at[dst_all.at[gi]], sem_s, add=True)

        def wait_scatter(gi, msg_v, sem_s):
            pltpu.make_async_copy(msg_v, accum.at[dst_all.at[gi]], sem_s).wait()

        ajslot = heads * chan
        dnums = lax.GatherDimensionNumbers(
            offset_dims=(), collapsed_slice_dims=(0,), start_index_map=(0,))

        def compute(rows_v, ai_v, msg_v):
            for e in range(G):
                ajv = rows_v[e, pl.ds(ajslot, 16)]
                aiv = ai_v[e, pl.ds(0, 16)]
                t = aiv + ajv
                sv = jnp.exp(jnp.maximum(t, 0.2 * t))
                msg_v[e, pl.ds(ajslot, 16)] = sv
                for k in range(ajslot // 16):
                    idxk = (lax.iota(jnp.int32, 16) + 16 * k) // chan
                    sek = lax.gather(
                        sv, idxk[:, None], dnums, (1,),
                        mode=lax.GatherScatterMode.PROMISE_IN_BOUNDS)
                    msg_v[e, pl.ds(16 * k, 16)] = (
                        rows_v[e, pl.ds(16 * k, 16)] * sek)

        issue(0, rows_a, ai_a, sem_ta, sem_aa)
        issue(1, rows_b, ai_b, sem_tb, sem_ab)
        start_scatter(0, msg_a, sem_sa)
        start_scatter(0, msg_b, sem_sb)

        def pair(p, carry):
            ga = 2 * p
            gb = 2 * p + 1
            wait_gathers(ga, rows_a, ai_a, sem_ta, sem_aa)
            wait_scatter(ga, msg_a, sem_sa)
            compute(rows_a, ai_a, msg_a)
            start_scatter(ga, msg_a, sem_sa)
            issue(ga + 2, rows_a, ai_a, sem_ta, sem_aa)
            wait_gathers(gb, rows_b, ai_b, sem_tb, sem_ab)
            wait_scatter(gb, msg_b, sem_sb)
            compute(rows_b, ai_b, msg_b)
            start_scatter(gb, msg_b, sem_sb)

            @pl.when(gb + 2 < GROUPS)
            def _():
                issue(gb + 2, rows_b, ai_b, sem_tb, sem_ab)

            return carry

        lax.fori_loop(0, (GROUPS - 1) // 2, pair, 0)
        glast = GROUPS - 1
        wait_gathers(glast, rows_a, ai_a, sem_ta, sem_aa)
        wait_scatter(glast - 2, msg_a, sem_sa)
        compute(rows_a, ai_a, msg_a)
        start_scatter(glast, msg_a, sem_sa)
        wait_scatter(glast, msg_a, sem_sa)
        wait_scatter(glast - 1, msg_b, sem_sb)
        plsc.subcore_barrier()
        pltpu.sync_copy(accum.at[pl.ds(z0, ZCHUNK)],
                        out_hbm.at[c, pl.ds(z0, ZCHUNK)])

        @pl.when(sid == 0)
        def _():
            pltpu.sync_copy(accum.at[pl.ds(ZCHUNK * SUBCORES, ZTAIL)],
                            out_hbm.at[c, pl.ds(ZCHUNK * SUBCORES, ZTAIL)])

    return edge_kernel


_EDGE1 = _make_sc_edge(R1, HEADS, HID)
_EDGE2 = _make_sc_edge(R2, 1, NCLS)


def kernel(x, edge_index, W1, att1, W2, att2):
    src = edge_index[0].reshape(CORES * SUBCORES * GROUPS, G)
    dst = edge_index[1].reshape(CORES * SUBCORES * GROUPS, G)
    eye = jnp.eye(HEADS, dtype=jnp.float32)
    ai_m1 = (att1[0, :, :HID][:, :, None] * eye[:, None, :]).reshape(HEADS * HID, HEADS)
    aj_m1 = (att1[0, :, HID:][:, :, None] * eye[:, None, :]).reshape(HEADS * HID, HEADS)
    rexp = jnp.repeat(eye, HID, axis=1)
    ai_m2 = att2[0, 0, :NCLS].reshape(NCLS, 1)
    aj_m2 = att2[0, 0, NCLS:].reshape(NCLS, 1)

    table1, aitab1, init1 = pl.pallas_call(
        _tc1_body,
        out_shape=(
            jax.ShapeDtypeStruct((N, R1), jnp.float32),
            jax.ShapeDtypeStruct((N, AIW), jnp.float32),
            jax.ShapeDtypeStruct((N, R1), jnp.float32),
        ),
    )(x, W1, ai_m1, aj_m1, rexp)

    z1 = jnp.zeros((N, R1), jnp.float32)
    partial1 = _EDGE1(table1, aitab1, src, dst, z1)

    table2, aitab2, init2 = pl.pallas_call(
        _tc2_body,
        out_shape=(
            jax.ShapeDtypeStruct((N, R2), jnp.float32),
            jax.ShapeDtypeStruct((N, AIW), jnp.float32),
            jax.ShapeDtypeStruct((N, R2), jnp.float32),
        ),
    )(partial1[0], partial1[1], init1, W2, ai_m2, aj_m2, rexp)

    z2 = jnp.zeros((N, R2), jnp.float32)
    partial2 = _EDGE2(table2, aitab2, src, dst, z2)

    return pl.pallas_call(
        _tc3_body,
        out_shape=jax.ShapeDtypeStruct((N, NCLS), jnp.float32),
    )(partial2[0], partial2[1], init2)


# speedup vs baseline: 173.7456x; 1.0027x over previous
import functools

import jax
import jax.numpy as jnp
from jax import lax
from jax.experimental import pallas as pl
from jax.experimental.pallas import tpu as pltpu
from jax.experimental.pallas import tpu_sc as plsc

N = 10000
E = 320000
HEADS = 8
HID = 8
NCLS = 16

CORES = 2
SUBCORES = 16
LANES = 16
G = 80
EPT = E // (CORES * SUBCORES)
GROUPS = EPT // G
ZCHUNK = 624
ZTAIL = N - ZCHUNK * SUBCORES
NBUF = 3

R1 = 80
R2 = 32
AIW = 16


def _tc1_body(x_ref, w_ref, ai_m_ref, aj_m_ref, rexp_ref,
              table_ref, aitab_ref, init_ref):
    h = jnp.dot(x_ref[...], w_ref[...], preferred_element_type=jnp.float32)
    ai = jnp.dot(h, ai_m_ref[...], preferred_element_type=jnp.float32)
    aj = jnp.dot(h, aj_m_ref[...], preferred_element_type=jnp.float32)
    t = ai + aj
    s = jnp.exp(jnp.maximum(t, 0.2 * t))
    se = jnp.dot(s, rexp_ref[...], preferred_element_type=jnp.float32)
    z8 = jnp.zeros((h.shape[0], 8), jnp.float32)
    table_ref[...] = jnp.concatenate([h, aj, z8], axis=1)
    aitab_ref[...] = jnp.concatenate([ai, z8], axis=1)
    init_ref[...] = jnp.concatenate([se * h, s, z8], axis=1)


def _tc2_body(p0_ref, p1_ref, init_ref_in, w2_ref, ai2_ref, aj2_ref, rexp_ref,
              table_ref, aitab_ref, init_ref):
    acc = p0_ref[...] + p1_ref[...] + init_ref_in[...]
    num = acc[:, :64]
    den = acc[:, 64:72]
    dene = jnp.dot(den, rexp_ref[...], preferred_element_type=jnp.float32)
    g = num / (dene + 1e-16)
    g = jnp.where(g > 0.0, g, jnp.exp(g) - 1.0)
    h2 = jnp.dot(g, w2_ref[...], preferred_element_type=jnp.float32)
    ai2 = jnp.dot(h2, ai2_ref[...], preferred_element_type=jnp.float32)
    aj2 = jnp.dot(h2, aj2_ref[...], preferred_element_type=jnp.float32)
    t = ai2 + aj2
    s = jnp.exp(jnp.maximum(t, 0.2 * t))
    z15 = jnp.zeros((acc.shape[0], 15), jnp.float32)
    table_ref[...] = jnp.concatenate([h2, aj2, z15], axis=1)
    aitab_ref[...] = jnp.concatenate([ai2, z15], axis=1)
    init_ref[...] = jnp.concatenate([h2 * s, s, z15], axis=1)


def _tc3_body(q0_ref, q1_ref, init_ref_in, out_ref):
    acc = q0_ref[...] + q1_ref[...] + init_ref_in[...]
    num = acc[:, :16]
    den = acc[:, 16:17]
    o = num / (den + 1e-16)
    m = jnp.max(o, axis=1, keepdims=True)
    ex = jnp.exp(o - m)
    lse = jnp.log(jnp.sum(ex, axis=1, keepdims=True))
    out_ref[...] = o - m - lse


def _make_sc_edge(rw, heads, chan):
    mesh = plsc.VectorSubcoreMesh(core_axis_name="c", subcore_axis_name="s")

    @functools.partial(
        pl.kernel,
        mesh=mesh,
        compiler_params=pltpu.CompilerParams(
            needs_layout_passes=False, use_tc_tiling_on_sc=False),
        out_type=jax.ShapeDtypeStruct((CORES, N, rw), jnp.float32),
        scratch_types=(
            [
                pltpu.VMEM((GROUPS, G), jnp.int32),
                pltpu.VMEM((GROUPS, G), jnp.int32),
            ]
            + [pltpu.VMEM((G, rw), jnp.float32) for _ in range(NBUF)]
            + [pltpu.VMEM((G, AIW), jnp.float32) for _ in range(NBUF)]
            + [pltpu.VMEM((G, rw), jnp.float32) for _ in range(NBUF)]
            + [pltpu.VMEM_SHARED((N, rw), jnp.float32)]
            + [pltpu.SemaphoreType.DMA] * (3 * NBUF)
        ),
    )
    def edge_kernel(table_hbm, aitab_hbm, srcg_hbm, dstg_hbm, zeros_hbm,
                    out_hbm, src_all, dst_all, *bufs):
        rows = bufs[0:NBUF]
        aib = bufs[NBUF:2 * NBUF]
        msg = bufs[2 * NBUF:3 * NBUF]
        accum = bufs[3 * NBUF]
        sem_t = bufs[3 * NBUF + 1:3 * NBUF + 1 + NBUF]
        sem_a = bufs[3 * NBUF + 1 + NBUF:3 * NBUF + 1 + 2 * NBUF]
        sem_s = bufs[3 * NBUF + 1 + 2 * NBUF:3 * NBUF + 1 + 3 * NBUF]
        c = lax.axis_index("c")
        sid = lax.axis_index("s")
        z0 = sid * ZCHUNK
        pltpu.sync_copy(zeros_hbm.at[pl.ds(z0, ZCHUNK)],
                        accum.at[pl.ds(z0, ZCHUNK)])

        @pl.when(sid == 0)
        def _():
            pltpu.sync_copy(zeros_hbm.at[pl.ds(ZCHUNK * SUBCORES, ZTAIL)],
                            accum.at[pl.ds(ZCHUNK * SUBCORES, ZTAIL)])

        gbase = (c * SUBCORES + sid) * GROUPS
        pltpu.sync_copy(srcg_hbm.at[pl.ds(gbase, GROUPS)], src_all)
        pltpu.sync_copy(dstg_hbm.at[pl.ds(gbase, GROUPS)], dst_all)
        for b in range(NBUF):
            pltpu.sync_copy(zeros_hbm.at[pl.ds(0, G)], msg[b])
        plsc.subcore_barrier()

        def issue(gi, b):
            pltpu.async_copy(table_hbm.at[src_all.at[gi]], rows[b], sem_t[b])
            pltpu.async_copy(aitab_hbm.at[dst_all.at[gi]], aib[b], sem_a[b])

        def wait_gathers(gi, b):
            pltpu.make_async_copy(table_hbm.at[src_all.at[gi]], rows[b], sem_t[b]).wait()
            pltpu.make_async_copy(aitab_hbm.at[dst_all.at[gi]], aib[b], sem_a[b]).wait()

        def start_scatter(gi, b):
            pltpu.async_copy(msg[b], accum.at[dst_all.at[gi]], sem_s[b], add=True)

        def wait_scatter(gi, b):
            pltpu.make_async_copy(msg[b], accum.at[dst_all.at[gi]], sem_s[b]).wait()

        ajslot = heads * chan
        dnums = lax.GatherDimensionNumbers(
            offset_dims=(), collapsed_slice_dims=(0,), start_index_map=(0,))

        def compute(rows_v, ai_v, msg_v):
            for e in range(G):
                ajv = rows_v[e, pl.ds(ajslot, 16)]
                aiv = ai_v[e, pl.ds(0, 16)]
                t = aiv + ajv
                sv = jnp.exp(jnp.maximum(t, 0.2 * t))
                msg_v[e, pl.ds(ajslot, 16)] = sv
                for k in range(ajslot // 16):
                    idxk = (lax.iota(jnp.int32, 16) + 16 * k) // chan
                    sek = lax.gather(
                        sv, idxk[:, None], dnums, (1,),
                        mode=lax.GatherScatterMode.PROMISE_IN_BOUNDS)
                    msg_v[e, pl.ds(16 * k, 16)] = (
                        rows_v[e, pl.ds(16 * k, 16)] * sek)

        for b in range(NBUF):
            issue(b, b)
            start_scatter(0, b)

        def round_(p, carry):
            for b in range(NBUF):
                g = NBUF * p + b
                wait_gathers(g, b)
                wait_scatter(g, b)
                compute(rows[b], aib[b], msg[b])
                start_scatter(g, b)

                @pl.when(g + NBUF < GROUPS)
                def _():
                    issue(g + NBUF, b)

            return carry

        nmain = GROUPS // NBUF
        lax.fori_loop(0, nmain, round_, 0)
        for r in range(GROUPS - nmain * NBUF):
            g = nmain * NBUF + r
            wait_gathers(g, r)
            wait_scatter(g, r)
            compute(rows[r], aib[r], msg[r])
            start_scatter(g, r)
        for b in range(NBUF):
            wait_scatter(0, b)
        plsc.subcore_barrier()
        pltpu.sync_copy(accum.at[pl.ds(z0, ZCHUNK)],
                        out_hbm.at[c, pl.ds(z0, ZCHUNK)])

        @pl.when(sid == 0)
        def _():
            pltpu.sync_copy(accum.at[pl.ds(ZCHUNK * SUBCORES, ZTAIL)],
                            out_hbm.at[c, pl.ds(ZCHUNK * SUBCORES, ZTAIL)])

    return edge_kernel


_EDGE1 = _make_sc_edge(R1, HEADS, HID)
_EDGE2 = _make_sc_edge(R2, 1, NCLS)


def kernel(x, edge_index, W1, att1, W2, att2):
    src = edge_index[0].reshape(CORES * SUBCORES * GROUPS, G)
    dst = edge_index[1].reshape(CORES * SUBCORES * GROUPS, G)
    eye = jnp.eye(HEADS, dtype=jnp.float32)
    ai_m1 = (att1[0, :, :HID][:, :, None] * eye[:, None, :]).reshape(HEADS * HID, HEADS)
    aj_m1 = (att1[0, :, HID:][:, :, None] * eye[:, None, :]).reshape(HEADS * HID, HEADS)
    rexp = jnp.repeat(eye, HID, axis=1)
    ai_m2 = att2[0, 0, :NCLS].reshape(NCLS, 1)
    aj_m2 = att2[0, 0, NCLS:].reshape(NCLS, 1)

    table1, aitab1, init1 = pl.pallas_call(
        _tc1_body,
        out_shape=(
            jax.ShapeDtypeStruct((N, R1), jnp.float32),
            jax.ShapeDtypeStruct((N, AIW), jnp.float32),
            jax.ShapeDtypeStruct((N, R1), jnp.float32),
        ),
    )(x, W1, ai_m1, aj_m1, rexp)

    z1 = jnp.zeros((N, R1), jnp.float32)
    partial1 = _EDGE1(table1, aitab1, src, dst, z1)

    table2, aitab2, init2 = pl.pallas_call(
        _tc2_body,
        out_shape=(
            jax.ShapeDtypeStruct((N, R2), jnp.float32),
            jax.ShapeDtypeStruct((N, AIW), jnp.float32),
            jax.ShapeDtypeStruct((N, R2), jnp.float32),
        ),
    )(partial1[0], partial1[1], init1, W2, ai_m2, aj_m2, rexp)

    z2 = jnp.zeros((N, R2), jnp.float32)
    partial2 = _EDGE2(table2, aitab2, src, dst, z2)

    return pl.pallas_call(
        _tc3_body,
        out_shape=jax.ShapeDtypeStruct((N, NCLS), jnp.float32),
    )(partial2[0], partial2[1], init2)
